# baseline (speedup 1.0000x reference)
import functools

import jax
import jax.numpy as jnp
from jax import lax
from jax.experimental import pallas as pl
from jax.experimental.pallas import tpu as pltpu
from jax.experimental.pallas import tpu_sc as plsc

TOKENS = 16384
N_EXPERTS = 64
TOPK = 8
N_GROUPS = 8
GROUP_SIZE = 8
TOPK_GROUPS = 4
ROUTE_SCALE = 2.5

NC = 2
NS = 16
NW = NC * NS
TOK_PER_W = TOKENS // NW
L = 16


def _score_kernel(x_ref, w_ref, o_ref):
    x = x_ref[...]
    w = w_ref[...]
    scores = jax.lax.dot_general(
        x, w, (((1,), (1,)), ((), ())), preferred_element_type=jnp.float32
    )
    o_ref[...] = jax.nn.sigmoid(scores)


def _scores_tc(x, weight):
    tokens, dim = x.shape
    tile_t = min(2048, tokens)
    return pl.pallas_call(
        _score_kernel,
        grid=(tokens // tile_t,),
        in_specs=[
            pl.BlockSpec((tile_t, dim), lambda i: (i, 0)),
            pl.BlockSpec((N_EXPERTS, dim), lambda i: (0, 0)),
        ],
        out_specs=pl.BlockSpec((tile_t, N_EXPERTS), lambda i: (i, 0)),
        out_shape=jax.ShapeDtypeStruct((tokens, N_EXPERTS), jnp.float32),
    )(x, weight)


def _g16(x, idx):
    return jax.lax.gather(
        x,
        idx[:, None],
        jax.lax.GatherDimensionNumbers(
            offset_dims=(), collapsed_slice_dims=(0,), start_index_map=(0,)
        ),
        (1,),
        mode=jax.lax.GatherScatterMode.PROMISE_IN_BOUNDS,
    )


@functools.partial(
    pl.kernel,
    mesh=plsc.VectorSubcoreMesh(core_axis_name="c", subcore_axis_name="s"),
    out_type=[
        jax.ShapeDtypeStruct((TOKENS, L), jnp.float32),
        jax.ShapeDtypeStruct((TOKENS, L), jnp.int32),
    ],
    scratch_types=[
        pltpu.VMEM((TOK_PER_W, N_EXPERTS), jnp.float32),
        pltpu.VMEM((TOK_PER_W, L), jnp.float32),
        pltpu.VMEM((TOK_PER_W, L), jnp.int32),
    ],
    compiler_params=pltpu.CompilerParams(needs_layout_passes=False, use_tc_tiling_on_sc=False),
)
def _route_sc(scores_hbm, wout_hbm, iout_hbm, s_v, w_v, i_v):
    wid = lax.axis_index("s") * NC + lax.axis_index("c")
    base = wid * TOK_PER_W
    pltpu.sync_copy(scores_hbm.at[pl.ds(base, TOK_PER_W)], s_v)

    lane = lax.iota(jnp.int32, L)
    neg1 = jnp.full((L,), -1.0, jnp.float32)
    pair_perm = (lane & 1) * GROUP_SIZE
    merge_perm = jnp.maximum(lane - 8, 0)
    lo8 = lane < 8

    def body(t, carry):
        s = [s_v[t, pl.ds(L * j, L)] for j in range(4)]
        gm = []
        for j in range(4):
            m = s[j]
            m = jnp.maximum(m, _g16(m, lane ^ 1))
            m = jnp.maximum(m, _g16(m, lane ^ 2))
            m = jnp.maximum(m, _g16(m, lane ^ 4))
            gm.append(m)
        p = [_g16(gm[j], pair_perm) for j in range(4)]
        g8 = jnp.where(lane < 2, p[0],
             jnp.where(lane < 4, _g16(p[1], lane - 2),
             jnp.where(lane < 6, _g16(p[2], lane - 4),
             jnp.where(lane < 8, _g16(p[3], jnp.maximum(lane - 6, 0)), neg1))))
        _, gsort = plsc.sort_key_val(g8, lane, descending=True)
        sel_g = [_g16(gsort, jnp.full((L,), k, jnp.int32)) for k in range(4)]
        ks, vs = [], []
        for j in range(4):
            grp = jnp.right_shift(lane, 3) + 2 * j
            selm = (grp == sel_g[0]) | (grp == sel_g[1]) | (grp == sel_g[2]) | (grp == sel_g[3])
            masked = jnp.where(selm, s[j], -1.0)
            kj, vj = plsc.sort_key_val(masked, lane + L * j, descending=True)
            ks.append(kj)
            vs.append(vj)

        def merge(ka, va, kb, vb):
            ck = jnp.where(lo8, ka, _g16(kb, merge_perm))
            cv = jnp.where(lo8, va, _g16(vb, merge_perm))
            return plsc.sort_key_val(ck, cv, descending=True)

        k01, v01 = merge(ks[0], vs[0], ks[1], vs[1])
        k23, v23 = merge(ks[2], vs[2], ks[3], vs[3])
        fk, fv = merge(k01, v01, k23, v23)

        k8 = jnp.where(lo8, fk, 0.0)
        tot = jnp.full((L,), jnp.sum(k8), jnp.float32)
        w_v[t, :] = k8 * ROUTE_SCALE / tot
        i_v[t, :] = jnp.where(lo8, fv, 0)
        return carry

    lax.fori_loop(0, TOK_PER_W, body, 0)
    pltpu.sync_copy(w_v, wout_hbm.at[pl.ds(base, TOK_PER_W)])
    pltpu.sync_copy(i_v, iout_hbm.at[pl.ds(base, TOK_PER_W)])


@jax.jit
def kernel(x, weight):
    scores = _scores_tc(x.astype(jnp.float32), weight.astype(jnp.float32))
    wfull, ifull = _route_sc(scores)
    return wfull[:, :TOPK], ifull[:, :TOPK]


# speedup vs baseline: 2.1107x; 2.1107x over previous
import functools

import jax
import jax.numpy as jnp
from jax.experimental import pallas as pl

N_EXPERTS = 64
TOPK = 8
N_GROUPS = 8
GROUP_SIZE = N_EXPERTS // N_GROUPS
TOPK_GROUPS = 4
ROUTE_SCALE = 2.5

NEG_INF = float("-inf")


def _gate_kernel(x_ref, w_ref, wout_ref, iout_ref):
    x = x_ref[...]
    w = w_ref[...]
    scores = jax.lax.dot_general(
        w, x, (((1,), (1,)), ((), ())), preferred_element_type=jnp.float32
    )
    scores = jax.nn.sigmoid(scores)
    t = scores.shape[1]

    erow = jax.lax.broadcasted_iota(jnp.int32, (N_EXPERTS, t), 0)
    grow8 = jax.lax.broadcasted_iota(jnp.int32, (N_GROUPS, t), 0)

    gmax = jnp.max(scores.reshape(N_GROUPS, GROUP_SIZE, t), axis=1)

    work = gmax
    sel8 = jnp.zeros((N_GROUPS, t), jnp.bool_)
    for _ in range(TOPK_GROUPS):
        m = jnp.max(work, axis=0, keepdims=True)
        cand = jnp.where(work == m, grow8, N_GROUPS)
        best_g = jnp.min(cand, axis=0, keepdims=True)
        pick = grow8 == best_g
        sel8 = jnp.logical_or(sel8, pick)
        work = jnp.where(pick, NEG_INF, work)

    sel = jnp.broadcast_to(sel8[:, None, :], (N_GROUPS, GROUP_SIZE, t)).reshape(
        N_EXPERTS, t
    )
    masked = jnp.where(sel, scores, NEG_INF)
    w_rows = []
    i_rows = []
    for _ in range(TOPK):
        m = jnp.max(masked, axis=0, keepdims=True)
        cand = jnp.where(masked == m, erow, N_EXPERTS)
        best = jnp.min(cand, axis=0, keepdims=True)
        w_rows.append(m)
        i_rows.append(best)
        masked = jnp.where(erow == best, NEG_INF, masked)
    wts = jnp.concatenate(w_rows, axis=0)
    idx = jnp.concatenate(i_rows, axis=0)
    wts = wts / jnp.sum(wts, axis=0, keepdims=True) * ROUTE_SCALE

    wout_ref[...] = wts.T
    iout_ref[...] = idx.T


@functools.partial(jax.jit, static_argnames=())
def kernel(x, weight):
    tokens, dim = x.shape
    tile_t = min(2048, tokens)
    grid = (tokens // tile_t,)
    wts, idx = pl.pallas_call(
        _gate_kernel,
        grid=grid,
        in_specs=[
            pl.BlockSpec((tile_t, dim), lambda i: (i, 0)),
            pl.BlockSpec((N_EXPERTS, dim), lambda i: (0, 0)),
        ],
        out_specs=[
            pl.BlockSpec((tile_t, TOPK), lambda i: (i, 0)),
            pl.BlockSpec((tile_t, TOPK), lambda i: (i, 0)),
        ],
        out_shape=[
            jax.ShapeDtypeStruct((tokens, TOPK), jnp.float32),
            jax.ShapeDtypeStruct((tokens, TOPK), jnp.int32),
        ],
    )(x.astype(jnp.float32), weight.astype(jnp.float32))
    return wts, idx


# speedup vs baseline: 2.1143x; 1.0017x over previous
import functools

import jax
import jax.numpy as jnp
from jax.experimental import pallas as pl

N_EXPERTS = 64
TOPK = 8
N_GROUPS = 8
GROUP_SIZE = N_EXPERTS // N_GROUPS
TOPK_GROUPS = 4
ROUTE_SCALE = 2.5

NEG_INF = float("-inf")


def _gate_kernel(x_ref, w_ref, wout_ref, iout_ref):
    x = x_ref[...]
    w = w_ref[...]
    scores = jax.lax.dot_general(
        w, x, (((1,), (1,)), ((), ())), preferred_element_type=jnp.float32
    )
    scores = jax.nn.sigmoid(scores)
    t = scores.shape[1]

    erow = jax.lax.broadcasted_iota(jnp.int32, (N_EXPERTS, t), 0)
    grow8 = jax.lax.broadcasted_iota(jnp.int32, (N_GROUPS, t), 0)

    gmax = jnp.max(scores.reshape(N_GROUPS, GROUP_SIZE, t), axis=1)

    work = gmax
    sel8 = jnp.zeros((N_GROUPS, t), jnp.bool_)
    for it in range(TOPK_GROUPS):
        m = jnp.max(work, axis=0, keepdims=True)
        cand = jnp.where(work == m, grow8, N_GROUPS)
        best_g = jnp.min(cand, axis=0, keepdims=True)
        pick = grow8 == best_g
        sel8 = jnp.logical_or(sel8, pick)
        if it + 1 < TOPK_GROUPS:
            work = jnp.where(pick, NEG_INF, work)

    sel = jnp.broadcast_to(sel8[:, None, :], (N_GROUPS, GROUP_SIZE, t)).reshape(
        N_EXPERTS, t
    )
    masked = jnp.where(sel, scores, NEG_INF)
    w_rows = []
    i_rows = []
    for it in range(TOPK):
        m = jnp.max(masked, axis=0, keepdims=True)
        cand = jnp.where(masked == m, erow, N_EXPERTS)
        best = jnp.min(cand, axis=0, keepdims=True)
        w_rows.append(m)
        i_rows.append(best)
        if it + 1 < TOPK:
            masked = jnp.where(erow == best, NEG_INF, masked)
    wts = jnp.concatenate(w_rows, axis=0)
    idx = jnp.concatenate(i_rows, axis=0)
    wts = wts / jnp.sum(wts, axis=0, keepdims=True) * ROUTE_SCALE

    wout_ref[...] = wts.T
    iout_ref[...] = idx.T


@functools.partial(jax.jit, static_argnames=())
def kernel(x, weight):
    tokens, dim = x.shape
    tile_t = min(2048, tokens)
    grid = (tokens // tile_t,)
    wts, idx = pl.pallas_call(
        _gate_kernel,
        grid=grid,
        in_specs=[
            pl.BlockSpec((tile_t, dim), lambda i: (i, 0)),
            pl.BlockSpec((N_EXPERTS, dim), lambda i: (0, 0)),
        ],
        out_specs=[
            pl.BlockSpec((tile_t, TOPK), lambda i: (i, 0)),
            pl.BlockSpec((tile_t, TOPK), lambda i: (i, 0)),
        ],
        out_shape=[
            jax.ShapeDtypeStruct((tokens, TOPK), jnp.float32),
            jax.ShapeDtypeStruct((tokens, TOPK), jnp.int32),
        ],
    )(x.astype(jnp.float32), weight.astype(jnp.float32))
    return wts, idx
